# Initial kernel scaffold; baseline (speedup 1.0000x reference)
import jax
import jax.numpy as jnp
from jax.experimental import pallas as pl


def kernel(token_ids, weight):
    raise NotImplementedError("write your pallas kernel here")


# speedup vs baseline: 7.6262x; 7.6262x over previous
import functools

import jax
import jax.numpy as jnp
from jax import lax
from jax.experimental import pallas as pl
from jax.experimental.pallas import tpu as pltpu
from jax.experimental.pallas import tpu_sc as plsc

_D = 128
_B_TOTAL = 16384 * 200
_NW = 32
_B_PER_W = _B_TOTAL // _NW
_CHUNK = 256
_NCHUNK = _B_PER_W // _CHUNK

_mesh = plsc.VectorSubcoreMesh(core_axis_name="c", subcore_axis_name="s")


@functools.partial(
    pl.kernel,
    mesh=_mesh,
    out_type=jax.ShapeDtypeStruct((_B_TOTAL, _D), jnp.float32),
    scratch_types=[
        pltpu.VMEM((_CHUNK,), jnp.int32),
        pltpu.VMEM((_CHUNK, _D), jnp.float32),
        pltpu.SemaphoreType.DMA,
    ],
)
def _gather_kernel(table_hbm, idx_hbm, out_hbm, idx_v, rows_v, sem):
    wid = lax.axis_index("s") * 2 + lax.axis_index("c")
    base = wid * _B_PER_W

    def body(g, carry):
        off = base + g * _CHUNK
        pltpu.sync_copy(idx_hbm.at[pl.ds(off, _CHUNK)], idx_v)
        pltpu.async_copy(table_hbm.at[idx_v], rows_v, sem).wait()
        pltpu.sync_copy(rows_v, out_hbm.at[pl.ds(off, _CHUNK)])
        return carry

    lax.fori_loop(0, _NCHUNK, body, 0)


def kernel(token_ids, weight):
    flat = token_ids.reshape(-1).astype(jnp.int32)
    out = _gather_kernel(weight, flat)
    return out.reshape(token_ids.shape + (_D,))


# speedup vs baseline: 10.8530x; 1.4231x over previous
import functools

import jax
import jax.numpy as jnp
from jax import lax
from jax.experimental import pallas as pl
from jax.experimental.pallas import tpu as pltpu
from jax.experimental.pallas import tpu_sc as plsc

_D = 128
_B_TOTAL = 16384 * 200
_NW = 32
_B_PER_W = _B_TOTAL // _NW
_CHUNK = 400
_NCHUNK = _B_PER_W // _CHUNK

_mesh = plsc.VectorSubcoreMesh(core_axis_name="c", subcore_axis_name="s")


@functools.partial(
    pl.kernel,
    mesh=_mesh,
    out_type=jax.ShapeDtypeStruct((_B_TOTAL, _D), jnp.float32),
    scratch_types=[
        pltpu.VMEM((_CHUNK,), jnp.int32),
        pltpu.VMEM((_CHUNK,), jnp.int32),
        pltpu.VMEM((_CHUNK, _D), jnp.float32),
        pltpu.VMEM((_CHUNK, _D), jnp.float32),
        pltpu.SemaphoreType.DMA,
        pltpu.SemaphoreType.DMA,
        pltpu.SemaphoreType.DMA,
        pltpu.SemaphoreType.DMA,
    ],
)
def _gather_kernel(table_hbm, idx_hbm, out_hbm,
                   idx0, idx1, rows0, rows1, g0, g1, o0, o1):
    idx_v = (idx0, idx1)
    rows_v = (rows0, rows1)
    gsem = (g0, g1)
    osem = (o0, o1)

    wid = lax.axis_index("s") * 2 + lax.axis_index("c")
    base = wid * _B_PER_W

    for b in range(2):
        pltpu.sync_copy(idx_hbm.at[pl.ds(base + b * _CHUNK, _CHUNK)], idx_v[b])
        pltpu.async_copy(table_hbm.at[idx_v[b]], rows_v[b], gsem[b])

    def body(g2, carry):
        for b in range(2):
            g = g2 * 2 + b
            off = base + g * _CHUNK
            pltpu.make_async_copy(
                table_hbm.at[idx_v[b]], rows_v[b], gsem[b]).wait()
            wcopy = pltpu.async_copy(
                rows_v[b], out_hbm.at[pl.ds(off, _CHUNK)], osem[b])
            pltpu.sync_copy(
                idx_hbm.at[pl.ds(off + 2 * _CHUNK, _CHUNK)], idx_v[b])
            wcopy.wait()
            pltpu.async_copy(table_hbm.at[idx_v[b]], rows_v[b], gsem[b])
        return carry

    lax.fori_loop(0, _NCHUNK // 2 - 1, body, 0)

    for b in range(2):
        g = _NCHUNK - 2 + b
        off = base + g * _CHUNK
        pltpu.make_async_copy(
            table_hbm.at[idx_v[b]], rows_v[b], gsem[b]).wait()
        pltpu.sync_copy(rows_v[b], out_hbm.at[pl.ds(off, _CHUNK)])


def kernel(token_ids, weight):
    flat = token_ids.reshape(-1).astype(jnp.int32)
    out = _gather_kernel(weight, flat)
    return out.reshape(token_ids.shape + (_D,))


# speedup vs baseline: 10.8610x; 1.0007x over previous
import functools

import jax
import jax.numpy as jnp
from jax import lax
from jax.experimental import pallas as pl
from jax.experimental.pallas import tpu as pltpu
from jax.experimental.pallas import tpu_sc as plsc

_D = 128
_B_TOTAL = 16384 * 200
_NW = 32
_B_PER_W = _B_TOTAL // _NW
_CHUNK = 200
_NCHUNK = _B_PER_W // _CHUNK
_NBUF = 4

_mesh = plsc.VectorSubcoreMesh(core_axis_name="c", subcore_axis_name="s")


@functools.partial(
    pl.kernel,
    mesh=_mesh,
    out_type=jax.ShapeDtypeStruct((_B_TOTAL, _D), jnp.float32),
    scratch_types=(
        [pltpu.VMEM((_CHUNK,), jnp.int32) for _ in range(_NBUF)]
        + [pltpu.VMEM((_CHUNK, _D), jnp.float32) for _ in range(_NBUF)]
        + [pltpu.SemaphoreType.DMA for _ in range(2 * _NBUF)]
    ),
)
def _gather_kernel(table_hbm, idx_hbm, out_hbm, *scratch):
    idx_v = scratch[:_NBUF]
    rows_v = scratch[_NBUF:2 * _NBUF]
    gsem = scratch[2 * _NBUF:3 * _NBUF]
    osem = scratch[3 * _NBUF:]

    wid = lax.axis_index("s") * 2 + lax.axis_index("c")
    base = wid * _B_PER_W

    for b in range(_NBUF):
        pltpu.sync_copy(idx_hbm.at[pl.ds(base + b * _CHUNK, _CHUNK)], idx_v[b])
        pltpu.async_copy(table_hbm.at[idx_v[b]], rows_v[b], gsem[b])

    def body(j, carry):
        for b in range(_NBUF):
            g = j * _NBUF + b
            off = base + g * _CHUNK
            pltpu.make_async_copy(
                table_hbm.at[idx_v[b]], rows_v[b], gsem[b]).wait()
            wcopy = pltpu.async_copy(
                rows_v[b], out_hbm.at[pl.ds(off, _CHUNK)], osem[b])
            pltpu.sync_copy(
                idx_hbm.at[pl.ds(off + _NBUF * _CHUNK, _CHUNK)], idx_v[b])
            wcopy.wait()
            pltpu.async_copy(table_hbm.at[idx_v[b]], rows_v[b], gsem[b])
        return carry

    lax.fori_loop(0, _NCHUNK // _NBUF - 1, body, 0)

    for b in range(_NBUF):
        g = _NCHUNK - _NBUF + b
        off = base + g * _CHUNK
        pltpu.make_async_copy(
            table_hbm.at[idx_v[b]], rows_v[b], gsem[b]).wait()
        pltpu.sync_copy(rows_v[b], out_hbm.at[pl.ds(off, _CHUNK)])


def kernel(token_ids, weight):
    flat = token_ids.reshape(-1).astype(jnp.int32)
    out = _gather_kernel(weight, flat)
    return out.reshape(token_ids.shape + (_D,))


# speedup vs baseline: 10.8691x; 1.0007x over previous
import functools

import jax
import jax.numpy as jnp
from jax import lax
from jax.experimental import pallas as pl
from jax.experimental.pallas import tpu as pltpu
from jax.experimental.pallas import tpu_sc as plsc

_D = 128
_B_TOTAL = 16384 * 200
_NW = 32
_B_PER_W = _B_TOTAL // _NW
_CHUNK = 400
_NCHUNK = _B_PER_W // _CHUNK
_NBUF = 2

_mesh = plsc.VectorSubcoreMesh(core_axis_name="c", subcore_axis_name="s")


@functools.partial(
    pl.kernel,
    mesh=_mesh,
    out_type=jax.ShapeDtypeStruct((_B_TOTAL, _D), jnp.float32),
    scratch_types=(
        [pltpu.VMEM((_CHUNK,), jnp.int32) for _ in range(_NBUF)]
        + [pltpu.VMEM((_CHUNK, _D), jnp.float32) for _ in range(_NBUF)]
        + [pltpu.SemaphoreType.DMA for _ in range(2 * _NBUF)]
    ),
)
def _gather_kernel(table_hbm, idx_hbm, out_hbm, *scratch):
    idx_v = scratch[:_NBUF]
    rows_v = scratch[_NBUF:2 * _NBUF]
    gsem = scratch[2 * _NBUF:3 * _NBUF]
    osem = scratch[3 * _NBUF:]

    wid = lax.axis_index("s") * 2 + lax.axis_index("c")
    base = wid * _B_PER_W

    for b in range(_NBUF):
        pltpu.sync_copy(idx_hbm.at[pl.ds(base + b * _CHUNK, _CHUNK)], idx_v[b])
        pltpu.async_copy(table_hbm.at[idx_v[b]], rows_v[b], gsem[b])

    def body(j, carry):
        for b in range(_NBUF):
            g = j * _NBUF + b
            off = base + g * _CHUNK
            pltpu.make_async_copy(
                table_hbm.at[idx_v[b]], rows_v[b], gsem[b]).wait()
            wcopy = pltpu.async_copy(
                rows_v[b], out_hbm.at[pl.ds(off, _CHUNK)], osem[b])
            pltpu.sync_copy(
                idx_hbm.at[pl.ds(off + _NBUF * _CHUNK, _CHUNK)], idx_v[b])
            wcopy.wait()
            pltpu.async_copy(table_hbm.at[idx_v[b]], rows_v[b], gsem[b])
        return carry

    lax.fori_loop(0, _NCHUNK // _NBUF - 1, body, 0)

    for b in range(_NBUF):
        g = _NCHUNK - _NBUF + b
        off = base + g * _CHUNK
        pltpu.make_async_copy(
            table_hbm.at[idx_v[b]], rows_v[b], gsem[b]).wait()
        pltpu.sync_copy(rows_v[b], out_hbm.at[pl.ds(off, _CHUNK)])


def kernel(token_ids, weight):
    flat = token_ids.reshape(-1).astype(jnp.int32)
    out = _gather_kernel(weight, flat)
    return out.reshape(token_ids.shape + (_D,))
